# Initial kernel scaffold; baseline (speedup 1.0000x reference)
import jax
import jax.numpy as jnp
from jax.experimental import pallas as pl


def kernel(edge_index, inputs, W0s, W0n, b0, W1s, W1n, b1, W2s, W2n, b2):
    raise NotImplementedError("write your pallas kernel here")


# speedup vs baseline: 3.2931x; 3.2931x over previous
import functools

import jax
import jax.numpy as jnp
from jax import lax
from jax.experimental import pallas as pl
from jax.experimental.pallas import tpu as pltpu
from jax.experimental.pallas import tpu_sc as plsc

N = 10000
E = 320000
D = 128
N_CLASSES = 40

NC = 2
NS = 16
NW = NC * NS
CH = 128
CPT = 80
E_PAD = NW * CPT * CH
N_PAD = 10240
RPT = N_PAD // NS

@functools.cache
def _mesh():
    return plsc.VectorSubcoreMesh(core_axis_name="c", subcore_axis_name="s",
                                  num_cores=NC, num_subcores=NS)


def _zero_rows(buf, nrows, ncols):
    zero16 = jnp.zeros((16,), jnp.float32)

    def zrow(r, _):
        for j in range(ncols // 16):
            buf[r, pl.ds(j * 16, 16)] = zero16
        return 0

    lax.fori_loop(0, nrows, zrow, 0)


def _sc_agg_body(h_hbm, srcm_hbm, dstm_hbm, out_hbm, sidx_v, didx_v, rows_v,
                 acc_sh, sem):
    c = lax.axis_index("c")
    s = lax.axis_index("s")
    wid = c * NS + s
    _zero_rows(rows_v, CH, D)
    def zacc(i, _):
        pltpu.sync_copy(rows_v, acc_sh.at[pl.ds(s * RPT + i * CH, CH)])
        return 0
    lax.fori_loop(0, RPT // CH, zacc, 0)
    plsc.subcore_barrier()

    pltpu.sync_copy(srcm_hbm.at[wid], sidx_v)
    pltpu.sync_copy(dstm_hbm.at[wid], didx_v)

    def body(j, _):
        pltpu.async_copy(h_hbm.at[sidx_v.at[j]], rows_v, sem).wait()
        pltpu.sync_copy(rows_v, acc_sh.at[didx_v.at[j]], add=True)
        return 0
    lax.fori_loop(0, CPT, body, 0)
    plsc.subcore_barrier()

    pltpu.sync_copy(acc_sh.at[pl.ds(s * RPT, RPT)],
                    out_hbm.at[c, pl.ds(s * RPT, RPT)])


@functools.cache
def _sc_agg():
    return pl.kernel(
        _sc_agg_body,
        out_type=jax.ShapeDtypeStruct((NC, N_PAD, D), jnp.float32),
        mesh=_mesh(),
        scratch_types=[
            pltpu.VMEM((CPT, CH), jnp.int32),
            pltpu.VMEM((CPT, CH), jnp.int32),
            pltpu.VMEM((CH, D), jnp.float32),
            pltpu.VMEM_SHARED((N_PAD, D), jnp.float32),
            pltpu.SemaphoreType.DMA,
        ],
    )


def _sc_deg_body(dstm_hbm, out_hbm, didx_v, ones_v, zeros_v, acc_sh):
    c = lax.axis_index("c")
    s = lax.axis_index("s")
    wid = c * NS + s

    def fill(i, _):
        zeros_v[pl.ds(i * 16, 16)] = jnp.zeros((16,), jnp.float32)
        return 0
    lax.fori_loop(0, RPT // 16, fill, 0)

    def fill1(i, _):
        ones_v[pl.ds(i * 16, 16)] = jnp.ones((16,), jnp.float32)
        return 0
    lax.fori_loop(0, CH // 16, fill1, 0)

    pltpu.sync_copy(zeros_v, acc_sh.at[pl.ds(s * RPT, RPT)])
    plsc.subcore_barrier()

    pltpu.sync_copy(dstm_hbm.at[wid], didx_v)

    def body(j, _):
        pltpu.sync_copy(ones_v, acc_sh.at[didx_v.at[j]], add=True)
        return 0
    lax.fori_loop(0, CPT, body, 0)
    plsc.subcore_barrier()

    pltpu.sync_copy(acc_sh.at[pl.ds(s * RPT, RPT)],
                    out_hbm.at[c, pl.ds(s * RPT, RPT)])


@functools.cache
def _sc_deg():
    return pl.kernel(
        _sc_deg_body,
        out_type=jax.ShapeDtypeStruct((NC, N_PAD), jnp.float32),
        mesh=_mesh(),
        scratch_types=[
            pltpu.VMEM((CPT, CH), jnp.int32),
            pltpu.VMEM((CH,), jnp.float32),
            pltpu.VMEM((RPT,), jnp.float32),
            pltpu.VMEM_SHARED((N_PAD,), jnp.float32),
        ],
    )

BLK = 1024


def _tc_combine_body(h_ref, acc_ref, deg_ref, ws_ref, wn_ref, b_ref, o_ref, *,
                     relu):
    deg = deg_ref[0] + deg_ref[1]
    inv = 1.0 / jnp.maximum(deg, 1.0)
    neigh = (acc_ref[0] + acc_ref[1]) * inv[:, None]
    o = (jnp.dot(h_ref[...], ws_ref[...], preferred_element_type=jnp.float32,
                 precision=lax.Precision.HIGHEST)
         + jnp.dot(neigh, wn_ref[...], preferred_element_type=jnp.float32,
                   precision=lax.Precision.HIGHEST)
         + b_ref[...])
    o_ref[...] = jnp.maximum(o, 0.0) if relu else o


def _tc_combine(h, acc, deg, Ws, Wn, b, relu):
    d_out = Ws.shape[1]
    grid = (N_PAD // BLK,)
    return pl.pallas_call(
        functools.partial(_tc_combine_body, relu=relu),
        grid=grid,
        in_specs=[
            pl.BlockSpec((BLK, D), lambda i: (i, 0)),
            pl.BlockSpec((NC, BLK, D), lambda i: (0, i, 0)),
            pl.BlockSpec((NC, BLK), lambda i: (0, i)),
            pl.BlockSpec((D, d_out), lambda i: (0, 0)),
            pl.BlockSpec((D, d_out), lambda i: (0, 0)),
            pl.BlockSpec((1, d_out), lambda i: (0, 0)),
        ],
        out_specs=pl.BlockSpec((BLK, d_out), lambda i: (i, 0)),
        out_shape=jax.ShapeDtypeStruct((N_PAD, d_out), jnp.float32),
    )(h, acc, deg, Ws, Wn, b)


def kernel(edge_index, inputs, W0s, W0n, b0, W1s, W1n, b1, W2s, W2n, b2):
    src = edge_index[0].astype(jnp.int32)
    dst = edge_index[1].astype(jnp.int32)
    src_p = jnp.concatenate(
        [src, jnp.zeros((E_PAD - E,), jnp.int32)]).reshape(NW, CPT, CH)
    dst_p = jnp.concatenate(
        [dst, jnp.full((E_PAD - E,), N, jnp.int32)]).reshape(NW, CPT, CH)

    x = jnp.pad(inputs, ((0, N_PAD - N), (0, 0)))
    W2s_p = jnp.pad(W2s, ((0, 0), (0, D - N_CLASSES)))
    W2n_p = jnp.pad(W2n, ((0, 0), (0, D - N_CLASSES)))
    b2_p = jnp.pad(b2, (0, D - N_CLASSES))

    deg = _sc_deg()(dst_p)

    agg = _sc_agg()(x, src_p, dst_p)
    h0 = _tc_combine(x, agg, deg, W0s, W0n, b0[None], relu=False)
    agg = _sc_agg()(h0, src_p, dst_p)
    h1 = _tc_combine(h0, agg, deg, W1s, W1n, b1[None], relu=True)
    agg = _sc_agg()(h1, src_p, dst_p)
    h2 = _tc_combine(h1, agg, deg, W1s, W1n, b1[None], relu=True)
    agg = _sc_agg()(h2, src_p, dst_p)
    out = _tc_combine(h2, agg, deg, W2s_p, W2n_p, b2_p[None], relu=False)

    return (out[:N, :N_CLASSES], h2[:N])


# speedup vs baseline: 3.8036x; 1.1550x over previous
import functools

import jax
import jax.numpy as jnp
from jax import lax
from jax.experimental import pallas as pl
from jax.experimental.pallas import tpu as pltpu
from jax.experimental.pallas import tpu_sc as plsc

N = 10000
E = 320000
D = 128
N_CLASSES = 40

NC = 2
NS = 16
NW = NC * NS
CH = 128
IG = 8
TOT_CH = 2560
E_PAD = TOT_CH * CH
N_PAD = 10240
RPT = N_PAD // NS

CPT0 = 112
CPT1 = 48


@functools.cache
def _mesh():
    return plsc.VectorSubcoreMesh(core_axis_name="c", subcore_axis_name="s",
                                  num_cores=NC, num_subcores=NS)


def _zero_rows(buf, nrows, ncols):
    zero16 = jnp.zeros((16,), jnp.float32)

    def zrow(r, _):
        for j in range(ncols // 16):
            buf[r, pl.ds(j * 16, 16)] = zero16
        return 0

    lax.fori_loop(0, nrows, zrow, 0)


def _sc_agg_body(h_hbm, src_hbm, dst_hbm, out_hbm, sg0, dg0, sg1, dg1,
                 rows_a, rows_b, acc_sh, sem_a, sem_b, sem_i):
    c = lax.axis_index("c")
    s = lax.axis_index("s")
    base = jnp.where(c == 0, s * CPT0, NS * CPT0 + s * CPT1)
    ngrp = jnp.where(c == 0, CPT0 // IG, CPT1 // IG)

    pltpu.sync_copy(src_hbm.at[pl.ds(base, IG)], sg0)
    pltpu.sync_copy(dst_hbm.at[pl.ds(base, IG)], dg0)
    pltpu.async_copy(h_hbm.at[sg0.at[0]], rows_a, sem_a)

    _zero_rows(rows_b, CH, D)

    def zacc(i, _):
        pltpu.sync_copy(rows_b, acc_sh.at[pl.ds(s * RPT + i * CH, CH)])
        return 0
    lax.fori_loop(0, RPT // CH, zacc, 0)
    plsc.subcore_barrier()

    sgs, dgs = (sg0, sg1), (dg0, dg1)
    rows, sems = (rows_a, rows_b), (sem_a, sem_b)

    def outer(i, _):
        for p in range(2):
            g = 2 * i + p
            gnext = jnp.minimum(g + 1, ngrp - 1)
            sb, db = sgs[p], dgs[p]
            snb, dnb = sgs[1 - p], dgs[1 - p]
            off = base + gnext * IG
            pltpu.async_copy(src_hbm.at[pl.ds(off, IG)], snb, sem_i)
            pltpu.async_copy(dst_hbm.at[pl.ds(off, IG)], dnb, sem_i)
            for k in range(IG):
                cur, nxt = rows[k & 1], rows[1 - (k & 1)]
                scur, snxt = sems[k & 1], sems[1 - (k & 1)]
                if k < IG - 1:
                    pltpu.async_copy(h_hbm.at[sb.at[k + 1]], nxt, snxt)
                else:
                    pltpu.make_async_copy(
                        src_hbm.at[pl.ds(0, IG)], snb, sem_i).wait()
                    pltpu.make_async_copy(
                        dst_hbm.at[pl.ds(0, IG)], dnb, sem_i).wait()
                    pltpu.async_copy(h_hbm.at[snb.at[0]], nxt, snxt)
                pltpu.make_async_copy(
                    h_hbm.at[pl.ds(0, CH)], cur, scur).wait()
                pltpu.sync_copy(cur, acc_sh.at[db.at[k]], add=True)
        return 0
    lax.fori_loop(0, ngrp // 2, outer, 0)
    pltpu.make_async_copy(h_hbm.at[pl.ds(0, CH)], rows_a, sem_a).wait()
    plsc.subcore_barrier()

    pltpu.sync_copy(acc_sh.at[pl.ds(s * RPT, RPT)],
                    out_hbm.at[c, pl.ds(s * RPT, RPT)])


@functools.cache
def _sc_agg():
    return pl.kernel(
        _sc_agg_body,
        out_type=jax.ShapeDtypeStruct((NC, N_PAD, D), jnp.float32),
        mesh=_mesh(),
        scratch_types=[
            pltpu.VMEM((IG, CH), jnp.int32),
            pltpu.VMEM((IG, CH), jnp.int32),
            pltpu.VMEM((IG, CH), jnp.int32),
            pltpu.VMEM((IG, CH), jnp.int32),
            pltpu.VMEM((CH, D), jnp.float32),
            pltpu.VMEM((CH, D), jnp.float32),
            pltpu.VMEM_SHARED((N_PAD, D), jnp.float32),
            pltpu.SemaphoreType.DMA,
            pltpu.SemaphoreType.DMA,
            pltpu.SemaphoreType.DMA,
        ],
    )


def _sc_deg_body(dstm_hbm, out_hbm, didx_v, ones_v, zeros_v, acc_sh):
    c = lax.axis_index("c")
    s = lax.axis_index("s")
    wid = c * NS + s
    cpt = TOT_CH // NW

    def fill(i, _):
        zeros_v[pl.ds(i * 16, 16)] = jnp.zeros((16,), jnp.float32)
        return 0
    lax.fori_loop(0, RPT // 16, fill, 0)

    def fill1(i, _):
        ones_v[pl.ds(i * 16, 16)] = jnp.ones((16,), jnp.float32)
        return 0
    lax.fori_loop(0, CH // 16, fill1, 0)

    pltpu.sync_copy(zeros_v, acc_sh.at[pl.ds(s * RPT, RPT)])
    plsc.subcore_barrier()

    pltpu.sync_copy(dstm_hbm.at[pl.ds(wid * cpt, cpt)], didx_v)

    def body(j, _):
        pltpu.sync_copy(ones_v, acc_sh.at[didx_v.at[j]], add=True)
        return 0
    lax.fori_loop(0, cpt, body, 0)
    plsc.subcore_barrier()

    pltpu.sync_copy(acc_sh.at[pl.ds(s * RPT, RPT)],
                    out_hbm.at[c, pl.ds(s * RPT, RPT)])


@functools.cache
def _sc_deg():
    return pl.kernel(
        _sc_deg_body,
        out_type=jax.ShapeDtypeStruct((NC, N_PAD), jnp.float32),
        mesh=_mesh(),
        scratch_types=[
            pltpu.VMEM((TOT_CH // NW, CH), jnp.int32),
            pltpu.VMEM((CH,), jnp.float32),
            pltpu.VMEM((RPT,), jnp.float32),
            pltpu.VMEM_SHARED((N_PAD,), jnp.float32),
        ],
    )


BLK = 1024


def _dot(a, b):
    return jnp.dot(a, b, preferred_element_type=jnp.float32,
                   precision=lax.Precision.HIGHEST)


def _tc_combine_body(h_ref, acc_ref, deg_ref, ws_ref, wn_ref, b_ref, o_ref, *,
                     relu):
    deg = deg_ref[0] + deg_ref[1]
    inv = 1.0 / jnp.maximum(deg, 1.0)
    neigh = (acc_ref[0] + acc_ref[1]) * inv[:, None]
    o = _dot(h_ref[...], ws_ref[...]) + _dot(neigh, wn_ref[...]) + b_ref[...]
    o_ref[...] = jnp.maximum(o, 0.0) if relu else o


def _tc_combine(h, acc, deg, Ws, Wn, b, relu):
    d_out = Ws.shape[1]
    return pl.pallas_call(
        functools.partial(_tc_combine_body, relu=relu),
        grid=(N_PAD // BLK,),
        in_specs=[
            pl.BlockSpec((BLK, D), lambda i: (i, 0)),
            pl.BlockSpec((NC, BLK, D), lambda i: (0, i, 0)),
            pl.BlockSpec((NC, BLK), lambda i: (0, i)),
            pl.BlockSpec((D, d_out), lambda i: (0, 0)),
            pl.BlockSpec((D, d_out), lambda i: (0, 0)),
            pl.BlockSpec((1, d_out), lambda i: (0, 0)),
        ],
        out_specs=pl.BlockSpec((BLK, d_out), lambda i: (i, 0)),
        out_shape=jax.ShapeDtypeStruct((N_PAD, d_out), jnp.float32),
    )(h, acc, deg, Ws, Wn, b)


def kernel(edge_index, inputs, W0s, W0n, b0, W1s, W1n, b1, W2s, W2n, b2):
    src = edge_index[0].astype(jnp.int32)
    dst = edge_index[1].astype(jnp.int32)
    src_p = jnp.concatenate(
        [src, jnp.zeros((E_PAD - E,), jnp.int32)]).reshape(TOT_CH, CH)
    dst_p = jnp.concatenate(
        [dst, jnp.full((E_PAD - E,), N, jnp.int32)]).reshape(TOT_CH, CH)

    x = jnp.pad(inputs, ((0, N_PAD - N), (0, 0)))
    W2s_p = jnp.pad(W2s, ((0, 0), (0, D - N_CLASSES)))
    W2n_p = jnp.pad(W2n, ((0, 0), (0, D - N_CLASSES)))
    b2_p = jnp.pad(b2, (0, D - N_CLASSES))

    deg = _sc_deg()(dst_p)

    agg = _sc_agg()(x, src_p, dst_p)
    h0 = _tc_combine(x, agg, deg, W0s, W0n, b0[None], relu=False)
    agg = _sc_agg()(h0, src_p, dst_p)
    h1 = _tc_combine(h0, agg, deg, W1s, W1n, b1[None], relu=True)
    agg = _sc_agg()(h1, src_p, dst_p)
    h2 = _tc_combine(h1, agg, deg, W1s, W1n, b1[None], relu=True)
    agg = _sc_agg()(h2, src_p, dst_p)
    out = _tc_combine(h2, agg, deg, W2s_p, W2n_p, b2_p[None], relu=False)

    return (out[:N, :N_CLASSES], h2[:N])


# speedup vs baseline: 10.2052x; 2.6831x over previous
import functools

import jax
import jax.numpy as jnp
from jax import lax
from jax.experimental import pallas as pl
from jax.experimental.pallas import tpu as pltpu
from jax.experimental.pallas import tpu_sc as plsc

N = 10000
E = 320000
D = 128
N_CLASSES = 40

NC = 2
NS = 16
NW = NC * NS
CH = 128
IG = 8
TOT_CH = 2560
E_PAD = TOT_CH * CH
N_PAD = 10240
RPT = N_PAD // NS

CPT0 = 112
CPT1 = 48


@functools.cache
def _mesh():
    return plsc.VectorSubcoreMesh(core_axis_name="c", subcore_axis_name="s",
                                  num_cores=NC, num_subcores=NS)


def _zero_rows(buf, nrows, ncols):
    zero16 = jnp.zeros((16,), jnp.float32)

    def zrow(r, _):
        for j in range(ncols // 16):
            buf[r, pl.ds(j * 16, 16)] = zero16
        return 0

    lax.fori_loop(0, nrows, zrow, 0)


def _sc_agg_body(h_hbm, src_hbm, dst_hbm, out_hbm, sg0, dg0, sg1, dg1,
                 rows_a, rows_b, acc_sh, sem_a, sem_b, sem_i):
    c = lax.axis_index("c")
    s = lax.axis_index("s")
    base = jnp.where(c == 0, s * CPT0, NS * CPT0 + s * CPT1)
    ngrp = jnp.where(c == 0, CPT0 // IG, CPT1 // IG)

    pltpu.sync_copy(src_hbm.at[pl.ds(base, IG)], sg0)
    pltpu.sync_copy(dst_hbm.at[pl.ds(base, IG)], dg0)
    pltpu.async_copy(h_hbm.at[sg0.at[0]], rows_a, sem_a)

    _zero_rows(rows_b, CH, D)

    def zacc(i, _):
        pltpu.sync_copy(rows_b, acc_sh.at[pl.ds(s * RPT + i * CH, CH)])
        return 0
    lax.fori_loop(0, RPT // CH, zacc, 0)
    plsc.subcore_barrier()

    sgs, dgs = (sg0, sg1), (dg0, dg1)
    rows, sems = (rows_a, rows_b), (sem_a, sem_b)

    def outer(i, _):
        for p in range(2):
            g = 2 * i + p
            gnext = jnp.minimum(g + 1, ngrp - 1)
            sb, db = sgs[p], dgs[p]
            snb, dnb = sgs[1 - p], dgs[1 - p]
            off = base + gnext * IG
            pltpu.async_copy(src_hbm.at[pl.ds(off, IG)], snb, sem_i)
            pltpu.async_copy(dst_hbm.at[pl.ds(off, IG)], dnb, sem_i)
            for k in range(IG):
                cur, nxt = rows[k & 1], rows[1 - (k & 1)]
                scur, snxt = sems[k & 1], sems[1 - (k & 1)]
                if k < IG - 1:
                    pltpu.async_copy(h_hbm.at[sb.at[k + 1]], nxt, snxt)
                else:
                    pltpu.make_async_copy(
                        src_hbm.at[pl.ds(0, IG)], snb, sem_i).wait()
                    pltpu.make_async_copy(
                        dst_hbm.at[pl.ds(0, IG)], dnb, sem_i).wait()
                    pltpu.async_copy(h_hbm.at[snb.at[0]], nxt, snxt)
                pltpu.make_async_copy(
                    h_hbm.at[pl.ds(0, CH)], cur, scur).wait()
                pltpu.sync_copy(cur, acc_sh.at[db.at[k]], add=True)
        return 0
    lax.fori_loop(0, ngrp // 2, outer, 0)
    pltpu.make_async_copy(h_hbm.at[pl.ds(0, CH)], rows_a, sem_a).wait()
    plsc.subcore_barrier()

    pltpu.sync_copy(acc_sh.at[pl.ds(s * RPT, RPT)],
                    out_hbm.at[c, pl.ds(s * RPT, RPT)])


@functools.cache
def _sc_agg():
    return pl.kernel(
        _sc_agg_body,
        out_type=jax.ShapeDtypeStruct((NC, N_PAD, D), jnp.float32),
        mesh=_mesh(),
        scratch_types=[
            pltpu.VMEM((IG, CH), jnp.int32),
            pltpu.VMEM((IG, CH), jnp.int32),
            pltpu.VMEM((IG, CH), jnp.int32),
            pltpu.VMEM((IG, CH), jnp.int32),
            pltpu.VMEM((CH, D), jnp.float32),
            pltpu.VMEM((CH, D), jnp.float32),
            pltpu.VMEM_SHARED((N_PAD, D), jnp.float32),
            pltpu.SemaphoreType.DMA,
            pltpu.SemaphoreType.DMA,
            pltpu.SemaphoreType.DMA,
        ],
    )


def _sc_deg_body(dstm_hbm, out_hbm, didx_v, ones_v, zeros_v, acc_sh):
    c = lax.axis_index("c")
    s = lax.axis_index("s")
    wid = c * NS + s
    cpt = TOT_CH // NW

    def fill(i, _):
        zeros_v[pl.ds(i * 16, 16)] = jnp.zeros((16,), jnp.float32)
        return 0
    lax.fori_loop(0, RPT // 16, fill, 0)

    def fill1(i, _):
        ones_v[pl.ds(i * 16, 16)] = jnp.ones((16,), jnp.float32)
        return 0
    lax.fori_loop(0, CH // 16, fill1, 0)

    pltpu.sync_copy(zeros_v, acc_sh.at[pl.ds(s * RPT, RPT)])
    plsc.subcore_barrier()

    pltpu.sync_copy(dstm_hbm.at[pl.ds(wid * cpt, cpt)], didx_v)

    def body(j, _):
        pltpu.sync_copy(ones_v, acc_sh.at[didx_v.at[j]], add=True)
        return 0
    lax.fori_loop(0, cpt, body, 0)
    plsc.subcore_barrier()

    pltpu.sync_copy(acc_sh.at[pl.ds(s * RPT, RPT)],
                    out_hbm.at[c, pl.ds(s * RPT, RPT)])


@functools.cache
def _sc_deg():
    return pl.kernel(
        _sc_deg_body,
        out_type=jax.ShapeDtypeStruct((NC, N_PAD), jnp.float32),
        mesh=_mesh(),
        scratch_types=[
            pltpu.VMEM((TOT_CH // NW, CH), jnp.int32),
            pltpu.VMEM((CH,), jnp.float32),
            pltpu.VMEM((RPT,), jnp.float32),
            pltpu.VMEM_SHARED((N_PAD,), jnp.float32),
        ],
    )


BLK = 1024


def _dot(a, b):
    return jnp.dot(a, b, preferred_element_type=jnp.float32,
                   precision=lax.Precision.HIGHEST)


def _tc_combine_body(h_ref, acc_ref, deg_ref, ws_ref, wn_ref, b_ref, o_ref, *,
                     relu):
    deg = deg_ref[0] + deg_ref[1]
    inv = 1.0 / jnp.maximum(deg, 1.0)
    neigh = (acc_ref[0] + acc_ref[1]) * inv[:, None]
    o = _dot(h_ref[...], ws_ref[...]) + _dot(neigh, wn_ref[...]) + b_ref[...]
    o_ref[...] = jnp.maximum(o, 0.0) if relu else o


def _tc_combine(h, acc, deg, Ws, Wn, b, relu):
    d_out = Ws.shape[1]
    return pl.pallas_call(
        functools.partial(_tc_combine_body, relu=relu),
        grid=(N_PAD // BLK,),
        in_specs=[
            pl.BlockSpec((BLK, D), lambda i: (i, 0)),
            pl.BlockSpec((NC, BLK, D), lambda i: (0, i, 0)),
            pl.BlockSpec((NC, BLK), lambda i: (0, i)),
            pl.BlockSpec((D, d_out), lambda i: (0, 0)),
            pl.BlockSpec((D, d_out), lambda i: (0, 0)),
            pl.BlockSpec((1, d_out), lambda i: (0, 0)),
        ],
        out_specs=pl.BlockSpec((BLK, d_out), lambda i: (i, 0)),
        out_shape=jax.ShapeDtypeStruct((N_PAD, d_out), jnp.float32),
    )(h, acc, deg, Ws, Wn, b)


def kernel(edge_index, inputs, W0s, W0n, b0, W1s, W1n, b1, W2s, W2n, b2):
    src = edge_index[0].astype(jnp.int32)
    dst = edge_index[1].astype(jnp.int32)
    pad_i = jnp.arange(E_PAD - E, dtype=jnp.int32)
    src_p = jnp.concatenate(
        [src, pad_i % N]).reshape(TOT_CH, CH)
    dst_p = jnp.concatenate(
        [dst, N + pad_i % (N_PAD - N)]).reshape(TOT_CH, CH)

    x = jnp.pad(inputs, ((0, N_PAD - N), (0, 0)))
    W2s_p = jnp.pad(W2s, ((0, 0), (0, D - N_CLASSES)))
    W2n_p = jnp.pad(W2n, ((0, 0), (0, D - N_CLASSES)))
    b2_p = jnp.pad(b2, (0, D - N_CLASSES))

    deg = _sc_deg()(dst_p)

    agg = _sc_agg()(x, src_p, dst_p)
    h0 = _tc_combine(x, agg, deg, W0s, W0n, b0[None], relu=False)
    agg = _sc_agg()(h0, src_p, dst_p)
    h1 = _tc_combine(h0, agg, deg, W1s, W1n, b1[None], relu=True)
    agg = _sc_agg()(h1, src_p, dst_p)
    h2 = _tc_combine(h1, agg, deg, W1s, W1n, b1[None], relu=True)
    agg = _sc_agg()(h2, src_p, dst_p)
    out = _tc_combine(h2, agg, deg, W2s_p, W2n_p, b2_p[None], relu=False)

    return (out[:N, :N_CLASSES], h2[:N])


# speedup vs baseline: 12.8347x; 1.2577x over previous
import functools

import jax
import jax.numpy as jnp
from jax import lax
from jax.experimental import pallas as pl
from jax.experimental.pallas import tpu as pltpu
from jax.experimental.pallas import tpu_sc as plsc

N = 10000
E = 320000
D = 128
N_CLASSES = 40

NC = 2
NS = 16
NW = NC * NS
CH = 128
IG = 8
TOT_CH = 2560
E_PAD = TOT_CH * CH
N_PAD = 10240
RPT = N_PAD // NS

CPT0 = 80
CPT1 = 80


@functools.cache
def _mesh():
    return plsc.VectorSubcoreMesh(core_axis_name="c", subcore_axis_name="s",
                                  num_cores=NC, num_subcores=NS)


def _zero_rows(buf, nrows, ncols):
    zero16 = jnp.zeros((16,), jnp.float32)

    def zrow(r, _):
        for j in range(ncols // 16):
            buf[r, pl.ds(j * 16, 16)] = zero16
        return 0

    lax.fori_loop(0, nrows, zrow, 0)


def _sc_agg_body(h_hbm, src_hbm, dst_hbm, out_hbm, sg0, dg0, sg1, dg1,
                 rows_a, rows_b, acc_sh, sem_a, sem_b, sem_i):
    c = lax.axis_index("c")
    s = lax.axis_index("s")
    base = jnp.where(c == 0, s * CPT0, NS * CPT0 + s * CPT1)
    ngrp = jnp.where(c == 0, CPT0 // IG, CPT1 // IG)

    pltpu.sync_copy(src_hbm.at[pl.ds(base, IG)], sg0)
    pltpu.sync_copy(dst_hbm.at[pl.ds(base, IG)], dg0)
    pltpu.async_copy(h_hbm.at[sg0.at[0]], rows_a, sem_a)

    _zero_rows(rows_b, CH, D)

    def zacc(i, _):
        pltpu.sync_copy(rows_b, acc_sh.at[pl.ds(s * RPT + i * CH, CH)])
        return 0
    lax.fori_loop(0, RPT // CH, zacc, 0)
    plsc.subcore_barrier()

    sgs, dgs = (sg0, sg1), (dg0, dg1)
    rows, sems = (rows_a, rows_b), (sem_a, sem_b)

    def outer(i, _):
        for p in range(2):
            g = 2 * i + p
            gnext = jnp.minimum(g + 1, ngrp - 1)
            sb, db = sgs[p], dgs[p]
            snb, dnb = sgs[1 - p], dgs[1 - p]
            off = base + gnext * IG
            pltpu.async_copy(src_hbm.at[pl.ds(off, IG)], snb, sem_i)
            pltpu.async_copy(dst_hbm.at[pl.ds(off, IG)], dnb, sem_i)
            for k in range(IG):
                cur, nxt = rows[k & 1], rows[1 - (k & 1)]
                scur, snxt = sems[k & 1], sems[1 - (k & 1)]
                if k < IG - 1:
                    pltpu.async_copy(h_hbm.at[sb.at[k + 1]], nxt, snxt)
                else:
                    pltpu.make_async_copy(
                        src_hbm.at[pl.ds(0, IG)], snb, sem_i).wait()
                    pltpu.make_async_copy(
                        dst_hbm.at[pl.ds(0, IG)], dnb, sem_i).wait()
                    pltpu.async_copy(h_hbm.at[snb.at[0]], nxt, snxt)
                pltpu.make_async_copy(
                    h_hbm.at[pl.ds(0, CH)], cur, scur).wait()
                pltpu.sync_copy(cur, acc_sh.at[db.at[k]], add=True)
        return 0
    lax.fori_loop(0, ngrp // 2, outer, 0)
    pltpu.make_async_copy(h_hbm.at[pl.ds(0, CH)], rows_a, sem_a).wait()
    plsc.subcore_barrier()

    pltpu.sync_copy(acc_sh.at[pl.ds(s * RPT, RPT)],
                    out_hbm.at[c, pl.ds(s * RPT, RPT)])


@functools.cache
def _sc_agg():
    return pl.kernel(
        _sc_agg_body,
        out_type=jax.ShapeDtypeStruct((NC, N_PAD, D), jnp.float32),
        mesh=_mesh(),
        scratch_types=[
            pltpu.VMEM((IG, CH), jnp.int32),
            pltpu.VMEM((IG, CH), jnp.int32),
            pltpu.VMEM((IG, CH), jnp.int32),
            pltpu.VMEM((IG, CH), jnp.int32),
            pltpu.VMEM((CH, D), jnp.float32),
            pltpu.VMEM((CH, D), jnp.float32),
            pltpu.VMEM_SHARED((N_PAD, D), jnp.float32),
            pltpu.SemaphoreType.DMA,
            pltpu.SemaphoreType.DMA,
            pltpu.SemaphoreType.DMA,
        ],
    )


def _sc_deg_body(dstm_hbm, out_hbm, didx_v, ones_v, zeros_v, acc_sh):
    c = lax.axis_index("c")
    s = lax.axis_index("s")
    wid = c * NS + s
    cpt = TOT_CH // NW

    def fill(i, _):
        zeros_v[pl.ds(i * 16, 16)] = jnp.zeros((16,), jnp.float32)
        return 0
    lax.fori_loop(0, RPT // 16, fill, 0)

    def fill1(i, _):
        ones_v[pl.ds(i * 16, 16)] = jnp.ones((16,), jnp.float32)
        return 0
    lax.fori_loop(0, CH // 16, fill1, 0)

    pltpu.sync_copy(zeros_v, acc_sh.at[pl.ds(s * RPT, RPT)])
    plsc.subcore_barrier()

    pltpu.sync_copy(dstm_hbm.at[pl.ds(wid * cpt, cpt)], didx_v)

    def body(j, _):
        pltpu.sync_copy(ones_v, acc_sh.at[didx_v.at[j]], add=True)
        return 0
    lax.fori_loop(0, cpt, body, 0)
    plsc.subcore_barrier()

    pltpu.sync_copy(acc_sh.at[pl.ds(s * RPT, RPT)],
                    out_hbm.at[c, pl.ds(s * RPT, RPT)])


@functools.cache
def _sc_deg():
    return pl.kernel(
        _sc_deg_body,
        out_type=jax.ShapeDtypeStruct((NC, N_PAD), jnp.float32),
        mesh=_mesh(),
        scratch_types=[
            pltpu.VMEM((TOT_CH // NW, CH), jnp.int32),
            pltpu.VMEM((CH,), jnp.float32),
            pltpu.VMEM((RPT,), jnp.float32),
            pltpu.VMEM_SHARED((N_PAD,), jnp.float32),
        ],
    )


BLK = 1024


def _dot(a, b):
    return jnp.dot(a, b, preferred_element_type=jnp.float32,
                   precision=lax.Precision.HIGHEST)


def _tc_combine_body(h_ref, acc_ref, deg_ref, ws_ref, wn_ref, b_ref, o_ref, *,
                     relu):
    deg = deg_ref[0] + deg_ref[1]
    inv = 1.0 / jnp.maximum(deg, 1.0)
    neigh = (acc_ref[0] + acc_ref[1]) * inv[:, None]
    o = _dot(h_ref[...], ws_ref[...]) + _dot(neigh, wn_ref[...]) + b_ref[...]
    o_ref[...] = jnp.maximum(o, 0.0) if relu else o


def _tc_combine(h, acc, deg, Ws, Wn, b, relu):
    d_out = Ws.shape[1]
    return pl.pallas_call(
        functools.partial(_tc_combine_body, relu=relu),
        grid=(N_PAD // BLK,),
        in_specs=[
            pl.BlockSpec((BLK, D), lambda i: (i, 0)),
            pl.BlockSpec((NC, BLK, D), lambda i: (0, i, 0)),
            pl.BlockSpec((NC, BLK), lambda i: (0, i)),
            pl.BlockSpec((D, d_out), lambda i: (0, 0)),
            pl.BlockSpec((D, d_out), lambda i: (0, 0)),
            pl.BlockSpec((1, d_out), lambda i: (0, 0)),
        ],
        out_specs=pl.BlockSpec((BLK, d_out), lambda i: (i, 0)),
        out_shape=jax.ShapeDtypeStruct((N_PAD, d_out), jnp.float32),
    )(h, acc, deg, Ws, Wn, b)


def kernel(edge_index, inputs, W0s, W0n, b0, W1s, W1n, b1, W2s, W2n, b2):
    src = edge_index[0].astype(jnp.int32)
    dst = edge_index[1].astype(jnp.int32)
    pad_i = jnp.arange(E_PAD - E, dtype=jnp.int32)
    src_p = jnp.concatenate(
        [src, pad_i % N]).reshape(TOT_CH, CH)
    dst_p = jnp.concatenate(
        [dst, N + pad_i % (N_PAD - N)]).reshape(TOT_CH, CH)

    x = jnp.pad(inputs, ((0, N_PAD - N), (0, 0)))
    W2s_p = jnp.pad(W2s, ((0, 0), (0, D - N_CLASSES)))
    W2n_p = jnp.pad(W2n, ((0, 0), (0, D - N_CLASSES)))
    b2_p = jnp.pad(b2, (0, D - N_CLASSES))

    deg = _sc_deg()(dst_p)

    agg = _sc_agg()(x, src_p, dst_p)
    h0 = _tc_combine(x, agg, deg, W0s, W0n, b0[None], relu=False)
    agg = _sc_agg()(h0, src_p, dst_p)
    h1 = _tc_combine(h0, agg, deg, W1s, W1n, b1[None], relu=True)
    agg = _sc_agg()(h1, src_p, dst_p)
    h2 = _tc_combine(h1, agg, deg, W1s, W1n, b1[None], relu=True)
    agg = _sc_agg()(h2, src_p, dst_p)
    out = _tc_combine(h2, agg, deg, W2s_p, W2n_p, b2_p[None], relu=False)

    return (out[:N, :N_CLASSES], h2[:N])
